# baseline (speedup 1.0000x reference)
import functools

import jax
import jax.numpy as jnp
from jax import lax
from jax.experimental import pallas as pl
from jax.experimental.pallas import tpu as pltpu
from jax.experimental.pallas import tpu_sc as plsc

N = 320000
D = 128
S = 512
NC = 2
NS = 16
NW = NC * NS
RPT = N // NW
CW = 128

CHA = 80
NCHA = RPT // CHA

CHC = 80
NCHC = RPT // CHC
NBC = 3


def _mesh():
    return plsc.VectorSubcoreMesh(core_axis_name="c", subcore_axis_name="s")


def _segment_partials_body(x_hbm, batch_hbm, zsum_hbm, zcnt_hbm, ones_hbm,
                           psum_hbm, pcnt_hbm,
                           rowbuf, idxa, idxb, ones, accs, accc,
                           semx0, semx1, semi0, semi1, sema0, sema1,
                           semc0, semc1):
    c = lax.axis_index("c")
    s = lax.axis_index("s")
    wid = s * NC + c
    base = wid * RPT

    pltpu.sync_copy(ones_hbm, ones)

    @pl.when(s == 0)
    def _init():
        pltpu.sync_copy(zsum_hbm, accs)
        pltpu.sync_copy(zcnt_hbm, accc)

    plsc.subcore_barrier()

    semx = (semx0, semx1)
    semi = (semi0, semi1)
    sema = (sema0, sema1)
    semc = (semc0, semc1)
    idxs = (idxa, idxb)

    def _issue_chunk(g, bi):
        pltpu.async_copy(batch_hbm.at[pl.ds(base + g * CHA, CHA)],
                         idxs[bi], semi[bi])
        pltpu.async_copy(x_hbm.at[pl.ds(base + g * CHA, CHA)],
                         rowbuf.at[bi], semx[bi])

    _issue_chunk(0, 0)

    def _branch(g, bi):
        oth = 1 - bi
        nxt = g + 1

        @pl.when(g >= 1)
        def _drain_prev_adds():
            pltpu.make_async_copy(rowbuf.at[oth], accs.at[idxs[oth]],
                                  sema[oth]).wait()
            pltpu.make_async_copy(ones, accc.at[idxs[oth]],
                                  semc[oth]).wait()

        @pl.when(nxt < NCHA)
        def _prefetch():
            _issue_chunk(nxt, oth)

        pltpu.make_async_copy(batch_hbm.at[pl.ds(base, CHA)],
                              idxs[bi], semi[bi]).wait()
        pltpu.make_async_copy(x_hbm.at[pl.ds(base, CHA)],
                              rowbuf.at[bi], semx[bi]).wait()
        pltpu.async_copy(rowbuf.at[bi], accs.at[idxs[bi]], sema[bi],
                         add=True)
        pltpu.async_copy(ones, accc.at[idxs[bi]], semc[bi], add=True)

    def chunk_body(g, carry):
        b = g % 2
        pl.when(b == 0)(lambda: _branch(g, 0))
        pl.when(b == 1)(lambda: _branch(g, 1))
        return carry

    lax.fori_loop(0, NCHA, chunk_body, 0)

    last = (NCHA - 1) % 2
    pltpu.make_async_copy(rowbuf.at[last], accs.at[idxs[last]],
                          sema[last]).wait()
    pltpu.make_async_copy(ones, accc.at[idxs[last]], semc[last]).wait()

    plsc.subcore_barrier()

    @pl.when(s == 0)
    def _export():
        pltpu.sync_copy(accs, psum_hbm.at[c])
        pltpu.sync_copy(accc, pcnt_hbm.at[c])


def _make_segment_partials(interpret=False):
    return functools.partial(
        pl.kernel,
        mesh=_mesh(),
        out_type=(
            jax.ShapeDtypeStruct((NC, S, D), jnp.float32),
            jax.ShapeDtypeStruct((NC, S, CW), jnp.float32),
        ),
        scratch_types=[
            pltpu.VMEM((2, CHA, D), jnp.float32),
            pltpu.VMEM((CHA,), jnp.int32),
            pltpu.VMEM((CHA,), jnp.int32),
            pltpu.VMEM((CHA, CW), jnp.float32),
            pltpu.VMEM_SHARED((S, D), jnp.float32),
            pltpu.VMEM_SHARED((S, CW), jnp.float32),
            pltpu.SemaphoreType.DMA,
            pltpu.SemaphoreType.DMA,
            pltpu.SemaphoreType.DMA,
            pltpu.SemaphoreType.DMA,
            pltpu.SemaphoreType.DMA,
            pltpu.SemaphoreType.DMA,
            pltpu.SemaphoreType.DMA,
            pltpu.SemaphoreType.DMA,
        ],
        interpret=interpret,
    )(_segment_partials_body)


_segment_partials = _make_segment_partials()


def _proj_body(psum_ref, pcnt_ref, w_ref, b_ref, out_ref):
    sums = jnp.sum(psum_ref[...], axis=0)
    cnts = jnp.sum(pcnt_ref[...], axis=0)[:, 0:1]
    mean = sums / jnp.maximum(cnts, 1.0)
    vn = lax.dot_general(mean, w_ref[...], (((1,), (1,)), ((), ())),
                         preferred_element_type=jnp.float32)
    out_ref[...] = vn + b_ref[...]


_project = pl.pallas_call(
    _proj_body,
    out_shape=jax.ShapeDtypeStruct((S, D), jnp.float32),
)


def _broadcast_add_body(x_hbm, batch_hbm, vn_hbm, out_hbm,
                        xbuf, vnbuf, outbuf, idxall,
                        semx0, semx1, semx2, semg0, semg1, semg2,
                        semo0, semo1, semo2):
    c = lax.axis_index("c")
    s = lax.axis_index("s")
    wid = s * NC + c
    base = wid * RPT

    pltpu.sync_copy(batch_hbm.at[pl.ds(base, RPT)], idxall)

    semx = (semx0, semx1, semx2)
    semg = (semg0, semg1, semg2)
    semo = (semo0, semo1, semo2)

    def _issue_chunk(g, bi):
        pltpu.async_copy(x_hbm.at[pl.ds(base + g * CHC, CHC)],
                         xbuf.at[bi], semx[bi])
        pltpu.async_copy(vn_hbm.at[idxall.at[pl.ds(g * CHC, CHC)]],
                         vnbuf.at[bi], semg[bi])

    _issue_chunk(0, 0)
    _issue_chunk(1, 1)

    def _branch(g, bi):
        pfb = (bi + 2) % NBC
        pf = g + 2

        @pl.when(pf < NCHC)
        def _prefetch():
            _issue_chunk(pf, pfb)

        @pl.when(g >= NBC)
        def _drain_old_out():
            pltpu.make_async_copy(
                outbuf.at[bi], out_hbm.at[pl.ds(base, CHC)], semo[bi]).wait()

        pltpu.make_async_copy(x_hbm.at[pl.ds(base, CHC)],
                              xbuf.at[bi], semx[bi]).wait()
        pltpu.make_async_copy(vn_hbm.at[idxall.at[pl.ds(0, CHC)]],
                              vnbuf.at[bi], semg[bi]).wait()

        @plsc.parallel_loop(0, CHC, unroll=4)
        def _row_body(i):
            for k in range(D // 16):
                sl = pl.ds(k * 16, 16)
                outbuf[bi, i, sl] = xbuf[bi, i, sl] + vnbuf[bi, i, sl]

        pltpu.async_copy(outbuf.at[bi], out_hbm.at[pl.ds(base + g * CHC, CHC)],
                         semo[bi])

    def chunk_body(g, carry):
        b = g % NBC
        for bi in range(NBC):
            pl.when(b == bi)(functools.partial(_branch, g, bi))
        return carry

    lax.fori_loop(0, NCHC, chunk_body, 0)

    for t in range(NBC):
        bi = (NCHC - 1 - t) % NBC
        pltpu.make_async_copy(outbuf.at[bi], out_hbm.at[pl.ds(base, CHC)],
                              semo[bi]).wait()


def _make_broadcast_add(interpret=False):
    return functools.partial(
        pl.kernel,
        mesh=_mesh(),
        out_type=jax.ShapeDtypeStruct((N, D), jnp.float32),
        scratch_types=[
            pltpu.VMEM((NBC, CHC, D), jnp.float32),
            pltpu.VMEM((NBC, CHC, D), jnp.float32),
            pltpu.VMEM((NBC, CHC, D), jnp.float32),
            pltpu.VMEM((RPT,), jnp.int32),
            pltpu.SemaphoreType.DMA,
            pltpu.SemaphoreType.DMA,
            pltpu.SemaphoreType.DMA,
            pltpu.SemaphoreType.DMA,
            pltpu.SemaphoreType.DMA,
            pltpu.SemaphoreType.DMA,
            pltpu.SemaphoreType.DMA,
            pltpu.SemaphoreType.DMA,
            pltpu.SemaphoreType.DMA,
        ],
        interpret=interpret,
    )(_broadcast_add_body)


_broadcast_add = _make_broadcast_add()


def kernel(x, batch, layer_idx, W, b):
    del layer_idx
    batch32 = batch.astype(jnp.int32)
    zsum = jnp.zeros((S, D), jnp.float32)
    zcnt = jnp.zeros((S, CW), jnp.float32)
    ones = jnp.ones((CHA, CW), jnp.float32)
    psum, pcnt = _segment_partials(x, batch32, zsum, zcnt, ones)
    vn = _project(psum, pcnt, W, b.reshape(1, D))
    x_out = _broadcast_add(x, batch32, vn)
    return (x_out, vn)


# speedup vs baseline: 2.6204x; 2.6204x over previous
import functools

import jax
import jax.numpy as jnp
from jax import lax
from jax.experimental import pallas as pl
from jax.experimental.pallas import tpu as pltpu
from jax.experimental.pallas import tpu_sc as plsc

N = 320000
D = 128
S = 512
NC = 2
NS = 16
NW = NC * NS
RPT = N // NW
CW = 128

CHA = 80
NCHA = RPT // CHA

CHC = 80
NCHC = RPT // CHC
NBC = 3


def _mesh():
    return plsc.VectorSubcoreMesh(core_axis_name="c", subcore_axis_name="s")


def _segment_partials_body(x_hbm, batch_hbm, zsum_hbm, zcnt_hbm, ones_hbm,
                           psum_hbm, pcnt_hbm,
                           rowbuf, idxa, idxb, ones, accs, accc,
                           semx0, semx1, semi0, semi1, sema0, sema1,
                           semc0, semc1):
    c = lax.axis_index("c")
    s = lax.axis_index("s")
    wid = s * NC + c
    base = wid * RPT

    pltpu.sync_copy(ones_hbm, ones)

    @pl.when(s == 0)
    def _init():
        pltpu.sync_copy(zsum_hbm, accs)
        pltpu.sync_copy(zcnt_hbm, accc)

    plsc.subcore_barrier()

    semx = (semx0, semx1)
    semi = (semi0, semi1)
    sema = (sema0, sema1)
    semc = (semc0, semc1)
    idxs = (idxa, idxb)

    def _issue_chunk(g, bi):
        pltpu.async_copy(batch_hbm.at[pl.ds(base + g * CHA, CHA)],
                         idxs[bi], semi[bi])
        pltpu.async_copy(x_hbm.at[pl.ds(base + g * CHA, CHA)],
                         rowbuf.at[bi], semx[bi])

    _issue_chunk(0, 0)

    def _branch(g, bi):
        oth = 1 - bi
        nxt = g + 1

        @pl.when(g >= 1)
        def _drain_prev_adds():
            pltpu.make_async_copy(rowbuf.at[oth], accs.at[idxs[oth]],
                                  sema[oth]).wait()
            pltpu.make_async_copy(ones, accc.at[idxs[oth]],
                                  semc[oth]).wait()

        @pl.when(nxt < NCHA)
        def _prefetch():
            _issue_chunk(nxt, oth)

        pltpu.make_async_copy(batch_hbm.at[pl.ds(base, CHA)],
                              idxs[bi], semi[bi]).wait()
        pltpu.make_async_copy(x_hbm.at[pl.ds(base, CHA)],
                              rowbuf.at[bi], semx[bi]).wait()
        pltpu.async_copy(rowbuf.at[bi], accs.at[idxs[bi]], sema[bi],
                         add=True)
        pltpu.async_copy(ones, accc.at[idxs[bi]], semc[bi], add=True)

    def chunk_body(g, carry):
        b = g % 2
        pl.when(b == 0)(lambda: _branch(g, 0))
        pl.when(b == 1)(lambda: _branch(g, 1))
        return carry

    lax.fori_loop(0, NCHA, chunk_body, 0)

    last = (NCHA - 1) % 2
    pltpu.make_async_copy(rowbuf.at[last], accs.at[idxs[last]],
                          sema[last]).wait()
    pltpu.make_async_copy(ones, accc.at[idxs[last]], semc[last]).wait()

    plsc.subcore_barrier()

    @pl.when(s == 0)
    def _export():
        pltpu.sync_copy(accs, psum_hbm.at[c])
        pltpu.sync_copy(accc, pcnt_hbm.at[c])


def _make_segment_partials(interpret=False):
    return functools.partial(
        pl.kernel,
        mesh=_mesh(),
        out_type=(
            jax.ShapeDtypeStruct((NC, S, D), jnp.float32),
            jax.ShapeDtypeStruct((NC, S, CW), jnp.float32),
        ),
        scratch_types=[
            pltpu.VMEM((2, CHA, D), jnp.float32),
            pltpu.VMEM((CHA,), jnp.int32),
            pltpu.VMEM((CHA,), jnp.int32),
            pltpu.VMEM((CHA, CW), jnp.float32),
            pltpu.VMEM_SHARED((S, D), jnp.float32),
            pltpu.VMEM_SHARED((S, CW), jnp.float32),
            pltpu.SemaphoreType.DMA,
            pltpu.SemaphoreType.DMA,
            pltpu.SemaphoreType.DMA,
            pltpu.SemaphoreType.DMA,
            pltpu.SemaphoreType.DMA,
            pltpu.SemaphoreType.DMA,
            pltpu.SemaphoreType.DMA,
            pltpu.SemaphoreType.DMA,
        ],
        interpret=interpret,
    )(_segment_partials_body)


_segment_partials = _make_segment_partials()


def _proj_body(psum_ref, pcnt_ref, w_ref, b_ref, out_ref):
    sums = jnp.sum(psum_ref[...], axis=0)
    cnts = jnp.sum(pcnt_ref[...], axis=0)[:, 0:1]
    mean = sums / jnp.maximum(cnts, 1.0)
    vn = lax.dot_general(mean, w_ref[...], (((1,), (1,)), ((), ())),
                         preferred_element_type=jnp.float32)
    out_ref[...] = vn + b_ref[...]


_project = pl.pallas_call(
    _proj_body,
    out_shape=jax.ShapeDtypeStruct((S, D), jnp.float32),
)


def _broadcast_add_body(x_hbm, batch_hbm, vn_hbm, out_hbm,
                        xbuf, vnbuf, outbuf, idxall, vn_spmem,
                        semx0, semx1, semx2, semg0, semg1, semg2,
                        semo0, semo1, semo2):
    c = lax.axis_index("c")
    s = lax.axis_index("s")
    wid = s * NC + c
    base = wid * RPT

    pltpu.sync_copy(batch_hbm.at[pl.ds(base, RPT)], idxall)

    @pl.when(s == 0)
    def _stage_vn():
        pltpu.sync_copy(vn_hbm, vn_spmem)

    plsc.subcore_barrier()

    semx = (semx0, semx1, semx2)
    semg = (semg0, semg1, semg2)
    semo = (semo0, semo1, semo2)

    def _issue_chunk(g, bi):
        pltpu.async_copy(x_hbm.at[pl.ds(base + g * CHC, CHC)],
                         xbuf.at[bi], semx[bi])
        pltpu.async_copy(vn_spmem.at[idxall.at[pl.ds(g * CHC, CHC)]],
                         vnbuf.at[bi], semg[bi])

    _issue_chunk(0, 0)
    _issue_chunk(1, 1)

    def _branch(g, bi):
        pfb = (bi + 2) % NBC
        pf = g + 2

        @pl.when(pf < NCHC)
        def _prefetch():
            _issue_chunk(pf, pfb)

        @pl.when(g >= NBC)
        def _drain_old_out():
            pltpu.make_async_copy(
                outbuf.at[bi], out_hbm.at[pl.ds(base, CHC)], semo[bi]).wait()

        pltpu.make_async_copy(x_hbm.at[pl.ds(base, CHC)],
                              xbuf.at[bi], semx[bi]).wait()
        pltpu.make_async_copy(vn_spmem.at[idxall.at[pl.ds(0, CHC)]],
                              vnbuf.at[bi], semg[bi]).wait()

        @plsc.parallel_loop(0, CHC, unroll=4)
        def _row_body(i):
            for k in range(D // 16):
                sl = pl.ds(k * 16, 16)
                outbuf[bi, i, sl] = xbuf[bi, i, sl] + vnbuf[bi, i, sl]

        pltpu.async_copy(outbuf.at[bi], out_hbm.at[pl.ds(base + g * CHC, CHC)],
                         semo[bi])

    def chunk_body(g, carry):
        b = g % NBC
        for bi in range(NBC):
            pl.when(b == bi)(functools.partial(_branch, g, bi))
        return carry

    lax.fori_loop(0, NCHC, chunk_body, 0)

    for t in range(NBC):
        bi = (NCHC - 1 - t) % NBC
        pltpu.make_async_copy(outbuf.at[bi], out_hbm.at[pl.ds(base, CHC)],
                              semo[bi]).wait()


def _make_broadcast_add(interpret=False):
    return functools.partial(
        pl.kernel,
        mesh=_mesh(),
        out_type=jax.ShapeDtypeStruct((N, D), jnp.float32),
        scratch_types=[
            pltpu.VMEM((NBC, CHC, D), jnp.float32),
            pltpu.VMEM((NBC, CHC, D), jnp.float32),
            pltpu.VMEM((NBC, CHC, D), jnp.float32),
            pltpu.VMEM((RPT,), jnp.int32),
            pltpu.VMEM_SHARED((S, D), jnp.float32),
            pltpu.SemaphoreType.DMA,
            pltpu.SemaphoreType.DMA,
            pltpu.SemaphoreType.DMA,
            pltpu.SemaphoreType.DMA,
            pltpu.SemaphoreType.DMA,
            pltpu.SemaphoreType.DMA,
            pltpu.SemaphoreType.DMA,
            pltpu.SemaphoreType.DMA,
            pltpu.SemaphoreType.DMA,
        ],
        interpret=interpret,
    )(_broadcast_add_body)


_broadcast_add = _make_broadcast_add()


def kernel(x, batch, layer_idx, W, b):
    del layer_idx
    batch32 = batch.astype(jnp.int32)
    zsum = jnp.zeros((S, D), jnp.float32)
    zcnt = jnp.zeros((S, CW), jnp.float32)
    ones = jnp.ones((CHA, CW), jnp.float32)
    psum, pcnt = _segment_partials(x, batch32, zsum, zcnt, ones)
    vn = _project(psum, pcnt, W, b.reshape(1, D))
    x_out = _broadcast_add(x, batch32, vn)
    return (x_out, vn)
